# baseline (speedup 1.0000x reference)
import functools

import jax
import jax.numpy as jnp
from jax import lax
from jax.experimental import pallas as pl
from jax.experimental.pallas import tpu as pltpu, tpu_sc as plsc

N = 10000
E = 320000
D = 128

NC = 2
NS = 16
L = 16

EPC = E // NC
EPT = EPC // NS
C = 96
NFULL = 104
CT = EPT - NFULL * C
NTRIPLE = 102
RPT = 624
RLAST = N - NS * RPT
ZFULL = RPT // C
ZREM = RPT - ZFULL * C


def _sc_agg_body(feat_hbm, ei_hbm, ev_hbm, out_hbm,
                 accum, cols_all, rowvt,
                 rowv0, rowv1, rowv2, evv0, evv1, evv2, gbuf0, gbuf1, gbuf2,
                 semg0, semg1, semg2, semr0, semr1, semr2,
                 seme0, seme1, seme2, sems0, sems1, sems2, semz):
    cid = lax.axis_index("c")
    sid = lax.axis_index("s")
    ebase = cid * EPC + sid * EPT

    gbufs = (gbuf0, gbuf1, gbuf2)
    rowvs = (rowv0, rowv1, rowv2)
    evvs = (evv0, evv1, evv2)
    semgs = (semg0, semg1, semg2)
    semrs = (semr0, semr1, semr2)
    semes = (seme0, seme1, seme2)
    semss = (sems0, sems1, sems2)

    def start_chunk(i, b):
        pltpu.async_copy(
            feat_hbm.at[cols_all.at[pl.ds(i * C, C)]], gbufs[b], semgs[b])
        pltpu.async_copy(ei_hbm.at[pl.ds(ebase + i * C, C)], rowvs[b], semrs[b])
        pltpu.async_copy(ev_hbm.at[pl.ds(ebase + i * C, C)], evvs[b], semes[b])

    def process(i, b):
        gbuf, rowv, evv = gbufs[b], rowvs[b], evvs[b]
        pltpu.make_async_copy(
            feat_hbm.at[cols_all.at[pl.ds(i * C, C)]], gbuf, semgs[b]).wait()
        pltpu.make_async_copy(
            ev_hbm.at[pl.ds(ebase + i * C, C)], evv, semes[b]).wait()

        @plsc.parallel_loop(0, C // L)
        def group(g):
            ev16 = evv[pl.ds(g * L, L)]
            for j in range(L):
                e = g * L + j
                splat = jnp.broadcast_to(ev16[j], (L,))
                for q in range(D // L):
                    sl = pl.ds(q * L, L)
                    gbuf[e, sl] = gbuf[e, sl] * splat
        pltpu.make_async_copy(
            ei_hbm.at[pl.ds(ebase + i * C, C)], rowv, semrs[b]).wait()
        pltpu.async_copy(gbuf, accum.at[rowv], semss[b], add=True)

    def wait_scatter(b):
        pltpu.make_async_copy(gbufs[b], accum.at[rowvs[b]], semss[b]).wait()


    def zrow(r, _):
        for q in range(D // L):
            gbuf2[r, pl.ds(q * L, L)] = jnp.zeros((L,), jnp.float32)
        return 0

    lax.fori_loop(0, C, zrow, 0)
    for k in range(ZFULL):
        pltpu.async_copy(gbuf2, accum.at[pl.ds(sid * RPT + k * C, C)], semz)
    pltpu.async_copy(gbuf2.at[pl.ds(0, ZREM)],
                     accum.at[pl.ds(sid * RPT + ZFULL * C, ZREM)], semz)

    @pl.when(sid == NS - 1)
    def _():
        pltpu.async_copy(gbuf2.at[pl.ds(0, RLAST)],
                         accum.at[pl.ds(NS * RPT, RLAST)], semz)

    pltpu.sync_copy(ei_hbm.at[pl.ds(E + ebase, EPT)], cols_all)

    start_chunk(0, 0)
    start_chunk(1, 1)

    for k in range(ZFULL):
        pltpu.make_async_copy(
            gbuf2, accum.at[pl.ds(sid * RPT + k * C, C)], semz).wait()
    pltpu.make_async_copy(
        gbuf2.at[pl.ds(0, ZREM)],
        accum.at[pl.ds(sid * RPT + ZFULL * C, ZREM)], semz).wait()

    @pl.when(sid == NS - 1)
    def _():
        pltpu.make_async_copy(
            gbuf2.at[pl.ds(0, RLAST)],
            accum.at[pl.ds(NS * RPT, RLAST)], semz).wait()

    plsc.subcore_barrier()


    @pl.loop(0, NTRIPLE, step=3)
    def triple(g):
        for b in range(3):
            i = g + b
            process(i, b)
            if b == 0:
                @pl.when(g >= 1)
                def _():
                    wait_scatter(2)
            else:
                wait_scatter(b - 1)
            start_chunk(i + 2, (b + 2) % 3)

    process(NTRIPLE, 0)
    wait_scatter(2)
    process(NTRIPLE + 1, 1)
    wait_scatter(0)

    tbase = ebase + NFULL * C
    pltpu.sync_copy(ei_hbm.at[pl.ds(tbase, CT)], rowvt)
    pltpu.sync_copy(ev_hbm.at[pl.ds(tbase, CT)], evv2.at[pl.ds(0, CT)])
    pltpu.async_copy(
        feat_hbm.at[cols_all.at[pl.ds(NFULL * C, CT)]],
        gbuf2.at[pl.ds(0, CT)], semg2)
    pltpu.make_async_copy(
        feat_hbm.at[cols_all.at[pl.ds(NFULL * C, CT)]],
        gbuf2.at[pl.ds(0, CT)], semg2).wait()
    ev16 = evv2[pl.ds(0, L)]
    for j in range(L):
        splat = jnp.broadcast_to(ev16[j], (L,))
        for q in range(D // L):
            sl = pl.ds(q * L, L)
            gbuf2[j, sl] = gbuf2[j, sl] * splat
    pltpu.sync_copy(gbuf2.at[pl.ds(0, CT)], accum.at[rowvt], add=True)
    wait_scatter(1)

    plsc.subcore_barrier()

    r0 = sid * RPT
    pltpu.sync_copy(accum.at[pl.ds(r0, RPT)], out_hbm.at[cid, pl.ds(r0, RPT)])

    @pl.when(sid == NS - 1)
    def _():
        pltpu.sync_copy(accum.at[pl.ds(NS * RPT, RLAST)],
                        out_hbm.at[cid, pl.ds(NS * RPT, RLAST)])


_sc_agg = functools.partial(
    pl.kernel,
    out_type=jax.ShapeDtypeStruct((NC, N, D), jnp.float32),
    mesh=plsc.VectorSubcoreMesh(core_axis_name="c", subcore_axis_name="s"),
    scratch_types=[
        pltpu.VMEM_SHARED((N, D), jnp.float32),
        pltpu.VMEM((EPT,), jnp.int32),
        pltpu.VMEM((CT,), jnp.int32),
        pltpu.VMEM((C,), jnp.int32),
        pltpu.VMEM((C,), jnp.int32),
        pltpu.VMEM((C,), jnp.int32),
        pltpu.VMEM((C,), jnp.float32),
        pltpu.VMEM((C,), jnp.float32),
        pltpu.VMEM((C,), jnp.float32),
        pltpu.VMEM((C, D), jnp.float32),
        pltpu.VMEM((C, D), jnp.float32),
        pltpu.VMEM((C, D), jnp.float32),
    ] + [pltpu.SemaphoreType.DMA] * 13,
)(_sc_agg_body)


def _dot3(a, w):
    ah = a.astype(jnp.bfloat16)
    al = (a - ah.astype(jnp.float32)).astype(jnp.bfloat16)
    wh = w.astype(jnp.bfloat16)
    wl = (w - wh.astype(jnp.float32)).astype(jnp.bfloat16)
    acc = jnp.dot(ah, wl, preferred_element_type=jnp.float32)
    acc = acc + jnp.dot(al, wh, preferred_element_type=jnp.float32)
    return acc + jnp.dot(ah, wh, preferred_element_type=jnp.float32)


def _tc_mlp_body(p0, p1, x, w1, b1, w2, b2, eps, sc, off, o):
    a = p0[0] + p1[0] + (1.0 + eps[0, 0]) * x[...]
    h = _dot3(a, w1[...]) + b1[...]
    h = jnp.maximum(h, 0.0)
    h = _dot3(h, w2[...]) + b2[...]
    h = jnp.maximum(h, 0.0)
    mean = jnp.mean(h, axis=1, keepdims=True)
    cen = h - mean
    var = jnp.mean(cen * cen, axis=1, keepdims=True) + 1e-10
    o[...] = cen * sc[...] * lax.rsqrt(var) + off[...]


BM = 2000


def _tc_mlp(parts, x, w1, b1, w2, b2, eps, sc, off):
    row_spec = pl.BlockSpec((BM, D), lambda i: (i, 0))
    p0_spec = pl.BlockSpec((1, BM, D), lambda i: (0, i, 0))
    p1_spec = pl.BlockSpec((1, BM, D), lambda i: (1, i, 0))
    full_spec = pl.BlockSpec((D, D), lambda i: (0, 0))
    vec_spec = pl.BlockSpec((1, D), lambda i: (0, 0))
    eps_spec = pl.BlockSpec((1, 1), lambda i: (0, 0))
    return pl.pallas_call(
        _tc_mlp_body,
        grid=(N // BM,),
        in_specs=[p0_spec, p1_spec, row_spec, full_spec, vec_spec,
                  full_spec, vec_spec, eps_spec, vec_spec, vec_spec],
        out_specs=row_spec,
        out_shape=jax.ShapeDtypeStruct((N, D), jnp.float32),
    )(parts, parts, x, w1, b1, w2, b2, eps, sc, off)


@jax.jit
def kernel(feat_in, edge_index, edge_values, W1, b1, W2, b2, eps, scale, offset):
    ei_flat = edge_index.reshape(2 * E)
    partials = _sc_agg(feat_in, ei_flat, edge_values)
    return _tc_mlp(
        partials, feat_in, W1, b1.reshape(1, D), W2,
        b2.reshape(1, D), eps.reshape(1, 1).astype(jnp.float32),
        scale.reshape(1, D), offset.reshape(1, D))


# speedup vs baseline: 1.1265x; 1.1265x over previous
import functools

import jax
import jax.numpy as jnp
from jax import lax
from jax.experimental import pallas as pl
from jax.experimental.pallas import tpu as pltpu, tpu_sc as plsc

N = 10000
E = 320000
D = 128

NC = 2
NS = 16
L = 16

EPC = E // NC
EPT = EPC // NS
C = 96
NFULL = 104
CT = EPT - NFULL * C
NTRIPLE = 102
RPT = 624
RLAST = N - NS * RPT
ZFULL = RPT // C
ZREM = RPT - ZFULL * C


def _sc_agg_body(feat_hbm, ei_hbm, ev_hbm, out_hbm,
                 accum, cols_all, rowvt,
                 rowv0, rowv1, rowv2, evv0, evv1, evv2, gbuf0, gbuf1, gbuf2,
                 semg0, semg1, semg2, semr0, semr1, semr2,
                 seme0, seme1, seme2, sems0, sems1, sems2, semz):
    cid = lax.axis_index("c")
    sid = lax.axis_index("s")
    ebase = cid * EPC + sid * EPT

    gbufs = (gbuf0, gbuf1, gbuf2)
    rowvs = (rowv0, rowv1, rowv2)
    evvs = (evv0, evv1, evv2)
    semgs = (semg0, semg1, semg2)
    semrs = (semr0, semr1, semr2)
    semes = (seme0, seme1, seme2)
    semss = (sems0, sems1, sems2)

    def start_chunk(i, b):
        pltpu.async_copy(
            feat_hbm.at[cols_all.at[pl.ds(i * C, C)]], gbufs[b], semgs[b])
        pltpu.async_copy(ei_hbm.at[pl.ds(ebase + i * C, C)], rowvs[b], semrs[b])
        pltpu.async_copy(ev_hbm.at[pl.ds(ebase + i * C, C)], evvs[b], semes[b])

    def process(i, b):
        gbuf, rowv, evv = gbufs[b], rowvs[b], evvs[b]
        pltpu.make_async_copy(
            feat_hbm.at[cols_all.at[pl.ds(i * C, C)]], gbuf, semgs[b]).wait()
        pltpu.make_async_copy(
            ev_hbm.at[pl.ds(ebase + i * C, C)], evv, semes[b]).wait()

        def group(g, _):
            ev16 = evv[pl.ds(g * L, L)]
            for j in range(L):
                e = g * L + j
                splat = jnp.broadcast_to(ev16[j], (L,))
                for q in range(D // L):
                    sl = pl.ds(q * L, L)
                    gbuf[e, sl] = gbuf[e, sl] * splat
            return 0

        lax.fori_loop(0, C // L, group, 0, unroll=2)
        pltpu.make_async_copy(
            ei_hbm.at[pl.ds(ebase + i * C, C)], rowv, semrs[b]).wait()
        pltpu.async_copy(gbuf, accum.at[rowv], semss[b], add=True)

    def wait_scatter(b):
        pltpu.make_async_copy(gbufs[b], accum.at[rowvs[b]], semss[b]).wait()


    def zrow(r, _):
        for q in range(D // L):
            gbuf2[r, pl.ds(q * L, L)] = jnp.zeros((L,), jnp.float32)
        return 0

    lax.fori_loop(0, C, zrow, 0)
    for k in range(ZFULL):
        pltpu.async_copy(gbuf2, accum.at[pl.ds(sid * RPT + k * C, C)], semz)
    pltpu.async_copy(gbuf2.at[pl.ds(0, ZREM)],
                     accum.at[pl.ds(sid * RPT + ZFULL * C, ZREM)], semz)

    @pl.when(sid == NS - 1)
    def _():
        pltpu.async_copy(gbuf2.at[pl.ds(0, RLAST)],
                         accum.at[pl.ds(NS * RPT, RLAST)], semz)

    pltpu.sync_copy(ei_hbm.at[pl.ds(E + ebase, EPT)], cols_all)

    start_chunk(0, 0)
    start_chunk(1, 1)

    for k in range(ZFULL):
        pltpu.make_async_copy(
            gbuf2, accum.at[pl.ds(sid * RPT + k * C, C)], semz).wait()
    pltpu.make_async_copy(
        gbuf2.at[pl.ds(0, ZREM)],
        accum.at[pl.ds(sid * RPT + ZFULL * C, ZREM)], semz).wait()

    @pl.when(sid == NS - 1)
    def _():
        pltpu.make_async_copy(
            gbuf2.at[pl.ds(0, RLAST)],
            accum.at[pl.ds(NS * RPT, RLAST)], semz).wait()

    plsc.subcore_barrier()


    @pl.loop(0, NTRIPLE, step=3)
    def triple(g):
        for b in range(3):
            i = g + b
            process(i, b)
            if b == 0:
                @pl.when(g >= 1)
                def _():
                    wait_scatter(2)
            else:
                wait_scatter(b - 1)
            start_chunk(i + 2, (b + 2) % 3)

    process(NTRIPLE, 0)
    wait_scatter(2)
    process(NTRIPLE + 1, 1)
    wait_scatter(0)

    tbase = ebase + NFULL * C
    pltpu.sync_copy(ei_hbm.at[pl.ds(tbase, CT)], rowvt)
    pltpu.sync_copy(ev_hbm.at[pl.ds(tbase, CT)], evv2.at[pl.ds(0, CT)])
    pltpu.async_copy(
        feat_hbm.at[cols_all.at[pl.ds(NFULL * C, CT)]],
        gbuf2.at[pl.ds(0, CT)], semg2)
    pltpu.make_async_copy(
        feat_hbm.at[cols_all.at[pl.ds(NFULL * C, CT)]],
        gbuf2.at[pl.ds(0, CT)], semg2).wait()
    ev16 = evv2[pl.ds(0, L)]
    for j in range(L):
        splat = jnp.broadcast_to(ev16[j], (L,))
        for q in range(D // L):
            sl = pl.ds(q * L, L)
            gbuf2[j, sl] = gbuf2[j, sl] * splat
    pltpu.sync_copy(gbuf2.at[pl.ds(0, CT)], accum.at[rowvt], add=True)
    wait_scatter(1)

    plsc.subcore_barrier()

    r0 = sid * RPT
    pltpu.sync_copy(accum.at[pl.ds(r0, RPT)], out_hbm.at[cid, pl.ds(r0, RPT)])

    @pl.when(sid == NS - 1)
    def _():
        pltpu.sync_copy(accum.at[pl.ds(NS * RPT, RLAST)],
                        out_hbm.at[cid, pl.ds(NS * RPT, RLAST)])


_sc_agg = functools.partial(
    pl.kernel,
    out_type=jax.ShapeDtypeStruct((NC, N, D), jnp.float32),
    mesh=plsc.VectorSubcoreMesh(core_axis_name="c", subcore_axis_name="s"),
    scratch_types=[
        pltpu.VMEM_SHARED((N, D), jnp.float32),
        pltpu.VMEM((EPT,), jnp.int32),
        pltpu.VMEM((CT,), jnp.int32),
        pltpu.VMEM((C,), jnp.int32),
        pltpu.VMEM((C,), jnp.int32),
        pltpu.VMEM((C,), jnp.int32),
        pltpu.VMEM((C,), jnp.float32),
        pltpu.VMEM((C,), jnp.float32),
        pltpu.VMEM((C,), jnp.float32),
        pltpu.VMEM((C, D), jnp.float32),
        pltpu.VMEM((C, D), jnp.float32),
        pltpu.VMEM((C, D), jnp.float32),
    ] + [pltpu.SemaphoreType.DMA] * 13,
)(_sc_agg_body)


def _dot3(a, w):
    ah = a.astype(jnp.bfloat16)
    al = (a - ah.astype(jnp.float32)).astype(jnp.bfloat16)
    wh = w.astype(jnp.bfloat16)
    wl = (w - wh.astype(jnp.float32)).astype(jnp.bfloat16)
    acc = jnp.dot(ah, wl, preferred_element_type=jnp.float32)
    acc = acc + jnp.dot(al, wh, preferred_element_type=jnp.float32)
    return acc + jnp.dot(ah, wh, preferred_element_type=jnp.float32)


def _tc_mlp_body(p0, p1, x, w1, b1, w2, b2, eps, sc, off, o):
    a = p0[0] + p1[0] + (1.0 + eps[0, 0]) * x[...]
    h = _dot3(a, w1[...]) + b1[...]
    h = jnp.maximum(h, 0.0)
    h = _dot3(h, w2[...]) + b2[...]
    h = jnp.maximum(h, 0.0)
    mean = jnp.mean(h, axis=1, keepdims=True)
    cen = h - mean
    var = jnp.mean(cen * cen, axis=1, keepdims=True) + 1e-10
    o[...] = cen * sc[...] * lax.rsqrt(var) + off[...]


BM = 2000


def _tc_mlp(parts, x, w1, b1, w2, b2, eps, sc, off):
    row_spec = pl.BlockSpec((BM, D), lambda i: (i, 0))
    p0_spec = pl.BlockSpec((1, BM, D), lambda i: (0, i, 0))
    p1_spec = pl.BlockSpec((1, BM, D), lambda i: (1, i, 0))
    full_spec = pl.BlockSpec((D, D), lambda i: (0, 0))
    vec_spec = pl.BlockSpec((1, D), lambda i: (0, 0))
    eps_spec = pl.BlockSpec((1, 1), lambda i: (0, 0))
    return pl.pallas_call(
        _tc_mlp_body,
        grid=(N // BM,),
        in_specs=[p0_spec, p1_spec, row_spec, full_spec, vec_spec,
                  full_spec, vec_spec, eps_spec, vec_spec, vec_spec],
        out_specs=row_spec,
        out_shape=jax.ShapeDtypeStruct((N, D), jnp.float32),
    )(parts, parts, x, w1, b1, w2, b2, eps, sc, off)


@jax.jit
def kernel(feat_in, edge_index, edge_values, W1, b1, W2, b2, eps, scale, offset):
    ei_flat = edge_index.reshape(2 * E)
    partials = _sc_agg(feat_in, ei_flat, edge_values)
    return _tc_mlp(
        partials, feat_in, W1, b1.reshape(1, D), W2,
        b2.reshape(1, D), eps.reshape(1, 1).astype(jnp.float32),
        scale.reshape(1, D), offset.reshape(1, D))


# speedup vs baseline: 1.1419x; 1.0136x over previous
import functools

import jax
import jax.numpy as jnp
from jax import lax
from jax.experimental import pallas as pl
from jax.experimental.pallas import tpu as pltpu, tpu_sc as plsc

N = 10000
E = 320000
D = 128

NC = 2
NS = 16
L = 16

EPC = E // NC
EPT = EPC // NS
C = 96
NFULL = 104
CT = EPT - NFULL * C
NTRIPLE = 102
RPT = 624
RLAST = N - NS * RPT
ZFULL = RPT // C
ZREM = RPT - ZFULL * C


def _sc_agg_body(feat_hbm, ei_hbm, ev_hbm, out_hbm,
                 accum, cols_all, rowvt,
                 rowv0, rowv1, rowv2, evv0, evv1, evv2, gbuf0, gbuf1, gbuf2,
                 semg0, semg1, semg2, semr0, semr1, semr2,
                 seme0, seme1, seme2, sems0, sems1, sems2, semz):
    cid = lax.axis_index("c")
    sid = lax.axis_index("s")
    ebase = cid * EPC + sid * EPT

    gbufs = (gbuf0, gbuf1, gbuf2)
    rowvs = (rowv0, rowv1, rowv2)
    evvs = (evv0, evv1, evv2)
    semgs = (semg0, semg1, semg2)
    semrs = (semr0, semr1, semr2)
    semes = (seme0, seme1, seme2)
    semss = (sems0, sems1, sems2)

    def start_chunk(i, b):
        pltpu.async_copy(
            feat_hbm.at[cols_all.at[pl.ds(i * C, C)]], gbufs[b], semgs[b])
        pltpu.async_copy(ei_hbm.at[pl.ds(ebase + i * C, C)], rowvs[b], semrs[b])
        pltpu.async_copy(ev_hbm.at[pl.ds(ebase + i * C, C)], evvs[b], semes[b])

    def process(i, b):
        gbuf, rowv, evv = gbufs[b], rowvs[b], evvs[b]
        pltpu.make_async_copy(
            feat_hbm.at[cols_all.at[pl.ds(i * C, C)]], gbuf, semgs[b]).wait()
        pltpu.make_async_copy(
            ev_hbm.at[pl.ds(ebase + i * C, C)], evv, semes[b]).wait()

        def group(g, _):
            ev16 = evv[pl.ds(g * L, L)]
            for j in range(L):
                e = g * L + j
                splat = jnp.broadcast_to(ev16[j], (L,))
                for q in range(D // L):
                    sl = pl.ds(q * L, L)
                    gbuf[e, sl] = gbuf[e, sl] * splat
            return 0

        lax.fori_loop(0, C // L, group, 0)
        pltpu.make_async_copy(
            ei_hbm.at[pl.ds(ebase + i * C, C)], rowv, semrs[b]).wait()
        pltpu.async_copy(gbuf, accum.at[rowv], semss[b], add=True)

    def wait_scatter(b):
        pltpu.make_async_copy(gbufs[b], accum.at[rowvs[b]], semss[b]).wait()


    def zrow(r, _):
        for q in range(D // L):
            gbuf2[r, pl.ds(q * L, L)] = jnp.zeros((L,), jnp.float32)
        return 0

    lax.fori_loop(0, C, zrow, 0)
    for k in range(ZFULL):
        pltpu.async_copy(gbuf2, accum.at[pl.ds(sid * RPT + k * C, C)], semz)
    pltpu.async_copy(gbuf2.at[pl.ds(0, ZREM)],
                     accum.at[pl.ds(sid * RPT + ZFULL * C, ZREM)], semz)

    @pl.when(sid == NS - 1)
    def _():
        pltpu.async_copy(gbuf2.at[pl.ds(0, RLAST)],
                         accum.at[pl.ds(NS * RPT, RLAST)], semz)

    pltpu.sync_copy(ei_hbm.at[pl.ds(E + ebase, EPT)], cols_all)

    start_chunk(0, 0)
    start_chunk(1, 1)

    for k in range(ZFULL):
        pltpu.make_async_copy(
            gbuf2, accum.at[pl.ds(sid * RPT + k * C, C)], semz).wait()
    pltpu.make_async_copy(
        gbuf2.at[pl.ds(0, ZREM)],
        accum.at[pl.ds(sid * RPT + ZFULL * C, ZREM)], semz).wait()

    @pl.when(sid == NS - 1)
    def _():
        pltpu.make_async_copy(
            gbuf2.at[pl.ds(0, RLAST)],
            accum.at[pl.ds(NS * RPT, RLAST)], semz).wait()

    plsc.subcore_barrier()


    @pl.loop(0, NTRIPLE, step=3)
    def triple(g):
        for b in range(3):
            i = g + b
            process(i, b)
            if b == 0:
                @pl.when(g >= 1)
                def _():
                    wait_scatter(2)
            else:
                wait_scatter(b - 1)
            start_chunk(i + 2, (b + 2) % 3)

    process(NTRIPLE, 0)
    wait_scatter(2)
    process(NTRIPLE + 1, 1)
    wait_scatter(0)

    tbase = ebase + NFULL * C
    pltpu.sync_copy(ei_hbm.at[pl.ds(tbase, CT)], rowvt)
    pltpu.sync_copy(ev_hbm.at[pl.ds(tbase, CT)], evv2.at[pl.ds(0, CT)])
    pltpu.async_copy(
        feat_hbm.at[cols_all.at[pl.ds(NFULL * C, CT)]],
        gbuf2.at[pl.ds(0, CT)], semg2)
    pltpu.make_async_copy(
        feat_hbm.at[cols_all.at[pl.ds(NFULL * C, CT)]],
        gbuf2.at[pl.ds(0, CT)], semg2).wait()
    ev16 = evv2[pl.ds(0, L)]
    for j in range(L):
        splat = jnp.broadcast_to(ev16[j], (L,))
        for q in range(D // L):
            sl = pl.ds(q * L, L)
            gbuf2[j, sl] = gbuf2[j, sl] * splat
    pltpu.sync_copy(gbuf2.at[pl.ds(0, CT)], accum.at[rowvt], add=True)
    wait_scatter(1)

    plsc.subcore_barrier()

    r0 = sid * RPT
    pltpu.sync_copy(accum.at[pl.ds(r0, RPT)], out_hbm.at[cid, pl.ds(r0, RPT)])

    @pl.when(sid == NS - 1)
    def _():
        pltpu.sync_copy(accum.at[pl.ds(NS * RPT, RLAST)],
                        out_hbm.at[cid, pl.ds(NS * RPT, RLAST)])


_sc_agg = functools.partial(
    pl.kernel,
    out_type=jax.ShapeDtypeStruct((NC, N, D), jnp.float32),
    mesh=plsc.VectorSubcoreMesh(core_axis_name="c", subcore_axis_name="s"),
    scratch_types=[
        pltpu.VMEM_SHARED((N, D), jnp.float32),
        pltpu.VMEM((EPT,), jnp.int32),
        pltpu.VMEM((CT,), jnp.int32),
        pltpu.VMEM((C,), jnp.int32),
        pltpu.VMEM((C,), jnp.int32),
        pltpu.VMEM((C,), jnp.int32),
        pltpu.VMEM((C,), jnp.float32),
        pltpu.VMEM((C,), jnp.float32),
        pltpu.VMEM((C,), jnp.float32),
        pltpu.VMEM((C, D), jnp.float32),
        pltpu.VMEM((C, D), jnp.float32),
        pltpu.VMEM((C, D), jnp.float32),
    ] + [pltpu.SemaphoreType.DMA] * 13,
)(_sc_agg_body)


def _dot3(a, w):
    ah = a.astype(jnp.bfloat16)
    al = (a - ah.astype(jnp.float32)).astype(jnp.bfloat16)
    wh = w.astype(jnp.bfloat16)
    wl = (w - wh.astype(jnp.float32)).astype(jnp.bfloat16)
    acc = jnp.dot(ah, wl, preferred_element_type=jnp.float32)
    acc = acc + jnp.dot(al, wh, preferred_element_type=jnp.float32)
    return acc + jnp.dot(ah, wh, preferred_element_type=jnp.float32)


def _tc_mlp_body(p0, p1, x, w1, b1, w2, b2, eps, sc, off, o):
    a = p0[0] + p1[0] + (1.0 + eps[0, 0]) * x[...]
    h = _dot3(a, w1[...]) + b1[...]
    h = jnp.maximum(h, 0.0)
    h = _dot3(h, w2[...]) + b2[...]
    h = jnp.maximum(h, 0.0)
    mean = jnp.mean(h, axis=1, keepdims=True)
    cen = h - mean
    var = jnp.mean(cen * cen, axis=1, keepdims=True) + 1e-10
    o[...] = cen * sc[...] * lax.rsqrt(var) + off[...]


BM = 2000


def _tc_mlp(parts, x, w1, b1, w2, b2, eps, sc, off):
    row_spec = pl.BlockSpec((BM, D), lambda i: (i, 0))
    p0_spec = pl.BlockSpec((1, BM, D), lambda i: (0, i, 0))
    p1_spec = pl.BlockSpec((1, BM, D), lambda i: (1, i, 0))
    full_spec = pl.BlockSpec((D, D), lambda i: (0, 0))
    vec_spec = pl.BlockSpec((1, D), lambda i: (0, 0))
    eps_spec = pl.BlockSpec((1, 1), lambda i: (0, 0))
    return pl.pallas_call(
        _tc_mlp_body,
        grid=(N // BM,),
        in_specs=[p0_spec, p1_spec, row_spec, full_spec, vec_spec,
                  full_spec, vec_spec, eps_spec, vec_spec, vec_spec],
        out_specs=row_spec,
        out_shape=jax.ShapeDtypeStruct((N, D), jnp.float32),
    )(parts, parts, x, w1, b1, w2, b2, eps, sc, off)


@jax.jit
def kernel(feat_in, edge_index, edge_values, W1, b1, W2, b2, eps, scale, offset):
    ei_flat = edge_index.reshape(2 * E)
    partials = _sc_agg(feat_in, ei_flat, edge_values)
    return _tc_mlp(
        partials, feat_in, W1, b1.reshape(1, D), W2,
        b2.reshape(1, D), eps.reshape(1, 1).astype(jnp.float32),
        scale.reshape(1, D), offset.reshape(1, D))
